# baseline (speedup 1.0000x reference)
import functools
import jax
import jax.numpy as jnp
from jax import lax
from jax.experimental import pallas as pl
from jax.experimental.pallas import tpu as pltpu
from jax.experimental.pallas import tpu_sc as plsc

N = 10000
E = 160000
NT = 16
NC = 2
NP = 10240
EB = 128
WB = 8
NBUF = 2
DEPTH = 2
EP = 163840
NBLK = EP // (NT * EB)
NBLK2 = EP // (NC * NT * EB)
STRIPE = NP // NT
DC = 128


def _edge_pipeline(nblk, h_ref, acc, src_t, dst_t, srcw, dstw, rows,
                   semg, semsc, semi):
    nw = nblk // WB

    def sidx(b):
        return srcw.at[(b // WB) % 2, b % WB]

    def didx(b):
        return dstw.at[(b // WB) % 2, b % WB]

    def gissue(b, s):
        pltpu.async_copy(h_ref.at[sidx(b)], rows[s], semg[s])

    def gwait(s):
        pltpu.make_async_copy(h_ref.at[srcw.at[0, 0]], rows[s],
                              semg[s]).wait()

    def scissue(b, s):
        pltpu.async_copy(rows[s], acc.at[didx(b)], semsc[s], add=True)

    def scwait(s):
        pltpu.make_async_copy(rows[s], acc.at[dstw.at[0, 0]],
                              semsc[s]).wait()

    pltpu.sync_copy(src_t.at[pl.ds(0, WB)], srcw.at[0])
    pltpu.sync_copy(dst_t.at[pl.ds(0, WB)], dstw.at[0])
    for j in range(DEPTH):
        gissue(j, j % NBUF)

    def window(w, carry):
        ws = (w + 1) % 2

        @pl.when(w + 1 < nw)
        def _():
            pltpu.async_copy(src_t.at[pl.ds((w + 1) * WB, WB)],
                             srcw.at[ws], semi[0])
            pltpu.async_copy(dst_t.at[pl.ds((w + 1) * WB, WB)],
                             dstw.at[ws], semi[1])
        for j in range(WB):
            b = w * WB + j
            s = j % NBUF
            sd = (j + DEPTH) % NBUF
            gwait(s)
            scissue(b, s)

            @pl.when(b + DEPTH >= NBUF)
            def _():
                scwait(sd)
            if j == WB - DEPTH or (WB < DEPTH + 1 and j == 0):
                @pl.when(w + 1 < nw)
                def _():
                    pltpu.make_async_copy(src_t.at[pl.ds(0, WB)],
                                          srcw.at[ws], semi[0]).wait()
                    pltpu.make_async_copy(dst_t.at[pl.ds(0, WB)],
                                          dstw.at[ws], semi[1]).wait()

            @pl.when(b + DEPTH < nblk)
            def _():
                gissue(b + DEPTH, sd)
        return carry
    lax.fori_loop(0, nw, window, 0)
    for bb in range(nblk - NBUF + DEPTH, nblk):
        scwait(bb % NBUF)


def _segsum_cols_body(cpc, h_hbm, base_hbm, src_hbm, dst_hbm, out_hbm,
                      acc, srcw, dstw, *bufs):
    rows = bufs[0:NBUF]
    semg = bufs[NBUF:2 * NBUF]
    semsc = bufs[2 * NBUF:3 * NBUF]
    semi = bufs[3 * NBUF:]
    cid = lax.axis_index("c")
    sid = lax.axis_index("s")
    s0 = sid * STRIPE
    for j in range(cpc):
        k = cid + NC * j
        pltpu.sync_copy(base_hbm.at[pl.ds(s0, STRIPE)],
                        acc.at[pl.ds(s0, STRIPE)])
        plsc.subcore_barrier()
        _edge_pipeline(NBLK, h_hbm.at[k], acc, src_hbm.at[sid],
                       dst_hbm.at[sid], srcw, dstw, rows, semg, semsc, semi)
        plsc.subcore_barrier()
        pltpu.sync_copy(acc.at[pl.ds(s0, STRIPE)],
                        out_hbm.at[k].at[pl.ds(s0, STRIPE)])
        plsc.subcore_barrier()


def _segsum_esplit_body(h_hbm, base_hbm, src_hbm, dst_hbm, out_hbm,
                        acc, srcw, dstw, *bufs):
    rows = bufs[0:NBUF]
    semg = bufs[NBUF:2 * NBUF]
    semsc = bufs[2 * NBUF:3 * NBUF]
    semi = bufs[3 * NBUF:]
    cid = lax.axis_index("c")
    sid = lax.axis_index("s")
    et = cid * NT + sid
    s0 = sid * STRIPE
    pltpu.sync_copy(base_hbm.at[cid].at[pl.ds(s0, STRIPE)],
                    acc.at[pl.ds(s0, STRIPE)])
    plsc.subcore_barrier()
    _edge_pipeline(NBLK2, h_hbm, acc, src_hbm.at[et], dst_hbm.at[et],
                   srcw, dstw, rows, semg, semsc, semi)
    plsc.subcore_barrier()
    pltpu.sync_copy(acc.at[pl.ds(s0, STRIPE)],
                    out_hbm.at[cid].at[pl.ds(s0, STRIPE)])


def _sc_mesh():
    return plsc.VectorSubcoreMesh(core_axis_name="c", subcore_axis_name="s")


def _sc_scratch():
    return [
        pltpu.VMEM_SHARED((NP, DC), jnp.float32),
        pltpu.VMEM((2, WB, EB), jnp.int32),
        pltpu.VMEM((2, WB, EB), jnp.int32),
    ] + [pltpu.VMEM((EB, DC), jnp.float32)] * NBUF \
      + [pltpu.SemaphoreType.DMA] * (2 * NBUF + 2)


def _make_segsum_cols(C):
    return pl.kernel(
        functools.partial(_segsum_cols_body, C // NC),
        out_type=jax.ShapeDtypeStruct((C, NP, DC), jnp.float32),
        mesh=_sc_mesh(),
        scratch_types=_sc_scratch(),
    )


def _make_segsum_esplit():
    return pl.kernel(
        _segsum_esplit_body,
        out_type=jax.ShapeDtypeStruct((NC, NP, DC), jnp.float32),
        mesh=_sc_mesh(),
        scratch_types=_sc_scratch(),
    )


BM = 256
BMM = 400


def _self_body(hs, ws, b, o):
    f32 = jnp.float32
    C = hs.shape[0]
    acc = b[...] + jnp.zeros((BM, ws.shape[1]), f32)
    for c in range(C):
        acc = acc + jnp.dot(hs[c], ws[c * 128:(c + 1) * 128, :],
                            preferred_element_type=f32)
    o[...] = acc


def _layer0_body(sw, ag, wn, o):
    f32 = jnp.float32
    s = (jnp.dot(ag[0], wn[0:128, :], preferred_element_type=f32)
         + jnp.dot(ag[1], wn[128:256, :], preferred_element_type=f32))
    h = jnp.maximum(s + sw[...], 0.0)
    o[0] = h[:, 0:128]
    o[1] = h[:, 128:256]
    o[2] = h[:, 256:384]
    o[3] = h[:, 384:512]


def _layer12_body(sw, ag, wn1, ws2, wn2, b2, s2o, m2o):
    f32 = jnp.float32
    acc = sw[...] + jnp.zeros((BM, 512), f32)
    for c in range(4):
        acc = acc + jnp.dot(ag[c], wn1[c * 128:(c + 1) * 128, :],
                            preferred_element_type=f32)
    hh = jnp.maximum(acc, 0.0)
    z = jnp.zeros((BM, 64), f32)
    s2 = jnp.dot(hh, ws2[...], preferred_element_type=f32) + b2[...]
    m2 = jnp.dot(hh, wn2[...], preferred_element_type=f32)
    s2o[0] = jnp.concatenate([s2, z], axis=1)
    s2o[1] = jnp.zeros((BM, 128), f32)
    m2o[...] = jnp.concatenate([m2, z], axis=1)


def _merge_body(p, o):
    o[...] = p[0, :, 0:64] + p[1, :, 0:64]


def _stack_spec(C, bm):
    return pl.BlockSpec((C, bm, DC), lambda i: (0, i, 0))


def _full_spec(shape):
    return pl.BlockSpec(shape, lambda i: tuple(0 for _ in shape))


def kernel(x, edge_index, features,
           W_self0, W_neigh0, b0,
           W_self1, W_neigh1, b1,
           W_self2, W_neigh2, b2):
    f32 = jnp.float32
    pad_e = EP - E
    src_p = jnp.concatenate([edge_index[0], jnp.zeros((pad_e,), jnp.int32)])
    dst_p = jnp.concatenate([edge_index[1],
                             jnp.full((pad_e,), NP - 1, jnp.int32)])
    src3 = src_p.reshape(NT, NBLK, EB)
    dst3 = dst_p.reshape(NT, NBLK, EB)
    src3b = src_p.reshape(NC * NT, NBLK2, EB)
    dst3b = dst_p.reshape(NC * NT, NBLK2, EB)

    x_pad = jnp.pad(x, ((0, NP - N), (0, 0)))
    xs = jnp.stack([x_pad[:, 0:128], x_pad[:, 128:256]])
    zbase = jnp.zeros((NP, DC), f32)

    grid = (NP // BM,)

    def self_mm(h_chunks, C, ws, b, dout):
        return pl.pallas_call(
            _self_body,
            grid=grid,
            in_specs=[_stack_spec(C, BM), _full_spec((C * 128, dout)),
                      _full_spec((1, dout))],
            out_specs=pl.BlockSpec((BM, dout), lambda i: (i, 0)),
            out_shape=jax.ShapeDtypeStruct((NP, dout), f32),
        )(h_chunks, ws, b.reshape(1, dout))

    agg0 = _make_segsum_cols(2)(xs, zbase, src3, dst3)
    xsw = self_mm(xs, 2, W_self0, b0, 512)

    h1 = pl.pallas_call(
        _layer0_body,
        grid=grid,
        in_specs=[pl.BlockSpec((BM, 512), lambda i: (i, 0)),
                  _stack_spec(2, BM), _full_spec((256, 512))],
        out_specs=_stack_spec(4, BM),
        out_shape=jax.ShapeDtypeStruct((4, NP, DC), f32),
    )(xsw, agg0, W_neigh0)

    agg1 = _make_segsum_cols(4)(h1, zbase, src3, dst3)
    h1w = self_mm(h1, 4, W_self1, b1, 512)

    s2st, m2 = pl.pallas_call(
        _layer12_body,
        grid=grid,
        in_specs=[pl.BlockSpec((BM, 512), lambda i: (i, 0)),
                  _stack_spec(4, BM), _full_spec((512, 512)),
                  _full_spec((512, 64)), _full_spec((512, 64)),
                  _full_spec((1, 64))],
        out_specs=[_stack_spec(2, BM), pl.BlockSpec((BM, DC), lambda i: (i, 0))],
        out_shape=[jax.ShapeDtypeStruct((2, NP, DC), f32),
                   jax.ShapeDtypeStruct((NP, DC), f32)],
    )(h1w, agg1, W_neigh1, W_self2, W_neigh2, b2.reshape(1, 64))

    parts = _make_segsum_esplit()(m2, s2st, src3b, dst3b)

    out = pl.pallas_call(
        _merge_body,
        grid=(N // BMM,),
        in_specs=[_stack_spec(2, BMM)],
        out_specs=pl.BlockSpec((BMM, 64), lambda i: (i, 0)),
        out_shape=jax.ShapeDtypeStruct((N, 64), f32),
    )(parts)
    return out


# speedup vs baseline: 1.1182x; 1.1182x over previous
import functools
import jax
import jax.numpy as jnp
from jax import lax
from jax.experimental import pallas as pl
from jax.experimental.pallas import tpu as pltpu
from jax.experimental.pallas import tpu_sc as plsc

N = 10000
E = 160000
NT = 16
NC = 2
NP = 10240
EB = 128
WB = 8
NBUF = 2
DEPTH = 2
EP = 163840
NBLK = EP // (NT * EB)
NBLK2 = EP // (NC * NT * EB)
STRIPE = NP // NT
DC = 128


def _edge_pipeline(nblk, h_ref, acc, src_t, dst_t, srcw, dstw, rows,
                   semg, semsc, semi):
    nw = nblk // WB

    def sidx(b):
        return srcw.at[(b // WB) % 2, b % WB]

    def didx(b):
        return dstw.at[(b // WB) % 2, b % WB]

    def gissue(b, s):
        pltpu.async_copy(h_ref.at[sidx(b)], rows[s], semg[s])

    def gwait(s):
        pltpu.make_async_copy(h_ref.at[srcw.at[0, 0]], rows[s],
                              semg[s]).wait()

    def scissue(b, s):
        pltpu.async_copy(rows[s], acc.at[didx(b)], semsc[s], add=True)

    def scwait(s):
        pltpu.make_async_copy(rows[s], acc.at[dstw.at[0, 0]],
                              semsc[s]).wait()

    pltpu.sync_copy(src_t.at[pl.ds(0, WB)], srcw.at[0])
    pltpu.sync_copy(dst_t.at[pl.ds(0, WB)], dstw.at[0])
    for j in range(DEPTH):
        gissue(j, j % NBUF)

    def window(w, carry):
        ws = (w + 1) % 2

        @pl.when(w + 1 < nw)
        def _():
            pltpu.async_copy(src_t.at[pl.ds((w + 1) * WB, WB)],
                             srcw.at[ws], semi[0])
            pltpu.async_copy(dst_t.at[pl.ds((w + 1) * WB, WB)],
                             dstw.at[ws], semi[1])
        for j in range(WB):
            b = w * WB + j
            s = j % NBUF
            sd = (j + DEPTH) % NBUF
            gwait(s)
            scissue(b, s)

            @pl.when(b + DEPTH >= NBUF)
            def _():
                scwait(sd)
            if j == WB - DEPTH or (WB < DEPTH + 1 and j == 0):
                @pl.when(w + 1 < nw)
                def _():
                    pltpu.make_async_copy(src_t.at[pl.ds(0, WB)],
                                          srcw.at[ws], semi[0]).wait()
                    pltpu.make_async_copy(dst_t.at[pl.ds(0, WB)],
                                          dstw.at[ws], semi[1]).wait()

            @pl.when(b + DEPTH < nblk)
            def _():
                gissue(b + DEPTH, sd)
        return carry
    lax.fori_loop(0, nw, window, 0)
    for bb in range(nblk - NBUF + DEPTH, nblk):
        scwait(bb % NBUF)


def _segsum_cols_body(cpc, h_hbm, base_hbm, src_hbm, dst_hbm, out_hbm,
                      acc, srcw, dstw, *bufs):
    rows = bufs[0:NBUF]
    semg = bufs[NBUF:2 * NBUF]
    semsc = bufs[2 * NBUF:3 * NBUF]
    semi = bufs[3 * NBUF:]
    cid = lax.axis_index("c")
    sid = lax.axis_index("s")
    s0 = sid * STRIPE
    for j in range(cpc):
        k = cid + NC * j
        pltpu.sync_copy(base_hbm.at[pl.ds(s0, STRIPE)],
                        acc.at[pl.ds(s0, STRIPE)])
        plsc.subcore_barrier()
        _edge_pipeline(NBLK, h_hbm.at[k], acc, src_hbm.at[sid],
                       dst_hbm.at[sid], srcw, dstw, rows, semg, semsc, semi)
        plsc.subcore_barrier()
        pltpu.sync_copy(acc.at[pl.ds(s0, STRIPE)],
                        out_hbm.at[k].at[pl.ds(s0, STRIPE)])
        plsc.subcore_barrier()


def _segsum_esplit_body(h_hbm, base_hbm, src_hbm, dst_hbm, out_hbm,
                        acc, srcw, dstw, *bufs):
    rows = bufs[0:NBUF]
    semg = bufs[NBUF:2 * NBUF]
    semsc = bufs[2 * NBUF:3 * NBUF]
    semi = bufs[3 * NBUF:]
    cid = lax.axis_index("c")
    sid = lax.axis_index("s")
    et = cid * NT + sid
    s0 = sid * STRIPE
    pltpu.sync_copy(base_hbm.at[cid].at[pl.ds(s0, STRIPE)],
                    acc.at[pl.ds(s0, STRIPE)])
    plsc.subcore_barrier()
    _edge_pipeline(NBLK2, h_hbm, acc, src_hbm.at[et], dst_hbm.at[et],
                   srcw, dstw, rows, semg, semsc, semi)
    plsc.subcore_barrier()
    pltpu.sync_copy(acc.at[pl.ds(s0, STRIPE)],
                    out_hbm.at[cid].at[pl.ds(s0, STRIPE)])


def _sc_mesh():
    return plsc.VectorSubcoreMesh(core_axis_name="c", subcore_axis_name="s")


def _sc_scratch():
    return [
        pltpu.VMEM_SHARED((NP, DC), jnp.float32),
        pltpu.VMEM((2, WB, EB), jnp.int32),
        pltpu.VMEM((2, WB, EB), jnp.int32),
    ] + [pltpu.VMEM((EB, DC), jnp.float32)] * NBUF \
      + [pltpu.SemaphoreType.DMA] * (2 * NBUF + 2)


def _make_segsum_cols(C):
    return pl.kernel(
        functools.partial(_segsum_cols_body, C // NC),
        out_type=jax.ShapeDtypeStruct((C, NP, DC), jnp.float32),
        mesh=_sc_mesh(),
        scratch_types=_sc_scratch(),
    )


def _make_segsum_esplit():
    return pl.kernel(
        _segsum_esplit_body,
        out_type=jax.ShapeDtypeStruct((NC, NP, DC), jnp.float32),
        mesh=_sc_mesh(),
        scratch_types=_sc_scratch(),
    )


BM = 256
BMM = 400


BF = jnp.bfloat16


def _bdot(a, w):
    return jax.lax.dot(a.astype(BF), w.astype(BF),
                       preferred_element_type=jnp.float32)


def _layer0_body(xs, ag, ws, wn, b, o):
    s = (_bdot(xs[0], ws[0:128, :]) + _bdot(xs[1], ws[128:256, :])
         + _bdot(ag[0], wn[0:128, :]) + _bdot(ag[1], wn[128:256, :]))
    h = jnp.maximum(s + b[...], 0.0)
    o[0] = h[:, 0:128]
    o[1] = h[:, 128:256]
    o[2] = h[:, 256:384]
    o[3] = h[:, 384:512]


def _layer12_body(hs, ag, ws1, wn1, b1, ws2, wn2, b2, s2o, m2o):
    f32 = jnp.float32
    acc = b1[...] + jnp.zeros((BM, 512), f32)
    for c in range(4):
        acc = acc + _bdot(hs[c], ws1[c * 128:(c + 1) * 128, :])
        acc = acc + _bdot(ag[c], wn1[c * 128:(c + 1) * 128, :])
    hh = jnp.maximum(acc, 0.0)
    z = jnp.zeros((BM, 64), f32)
    s2 = _bdot(hh, ws2[...]) + b2[...]
    m2 = _bdot(hh, wn2[...])
    s2o[0] = jnp.concatenate([s2, z], axis=1)
    s2o[1] = jnp.zeros((BM, 128), f32)
    m2o[...] = jnp.concatenate([m2, z], axis=1)


def _merge_body(p, o):
    o[...] = p[0, :, 0:64] + p[1, :, 0:64]


def _stack_spec(C, bm):
    return pl.BlockSpec((C, bm, DC), lambda i: (0, i, 0))


def _full_spec(shape):
    return pl.BlockSpec(shape, lambda i: tuple(0 for _ in shape))


def kernel(x, edge_index, features,
           W_self0, W_neigh0, b0,
           W_self1, W_neigh1, b1,
           W_self2, W_neigh2, b2):
    f32 = jnp.float32
    pad_e = EP - E
    src_p = jnp.concatenate([edge_index[0], jnp.zeros((pad_e,), jnp.int32)])
    dst_p = jnp.concatenate([edge_index[1],
                             jnp.full((pad_e,), NP - 1, jnp.int32)])
    src3 = src_p.reshape(NT, NBLK, EB)
    dst3 = dst_p.reshape(NT, NBLK, EB)
    src3b = src_p.reshape(NC * NT, NBLK2, EB)
    dst3b = dst_p.reshape(NC * NT, NBLK2, EB)

    x_pad = jnp.pad(x, ((0, NP - N), (0, 0)))
    xs = jnp.stack([x_pad[:, 0:128], x_pad[:, 128:256]])
    zbase = jnp.zeros((NP, DC), f32)

    agg0 = _make_segsum_cols(2)(xs, zbase, src3, dst3)

    grid = (NP // BM,)
    h1 = pl.pallas_call(
        _layer0_body,
        grid=grid,
        in_specs=[_stack_spec(2, BM), _stack_spec(2, BM),
                  _full_spec((256, 512)), _full_spec((256, 512)),
                  _full_spec((1, 512))],
        out_specs=_stack_spec(4, BM),
        out_shape=jax.ShapeDtypeStruct((4, NP, DC), f32),
    )(xs, agg0, W_self0, W_neigh0, b0.reshape(1, 512))

    agg1 = _make_segsum_cols(4)(h1, zbase, src3, dst3)

    s2st, m2 = pl.pallas_call(
        _layer12_body,
        grid=grid,
        in_specs=[_stack_spec(4, BM), _stack_spec(4, BM),
                  _full_spec((512, 512)), _full_spec((512, 512)),
                  _full_spec((1, 512)),
                  _full_spec((512, 64)), _full_spec((512, 64)),
                  _full_spec((1, 64))],
        out_specs=[_stack_spec(2, BM), pl.BlockSpec((BM, DC), lambda i: (i, 0))],
        out_shape=[jax.ShapeDtypeStruct((2, NP, DC), f32),
                   jax.ShapeDtypeStruct((NP, DC), f32)],
    )(h1, agg1, W_self1, W_neigh1, b1.reshape(1, 512),
      W_self2, W_neigh2, b2.reshape(1, 64))

    parts = _make_segsum_esplit()(m2, s2st, src3b, dst3b)

    out = pl.pallas_call(
        _merge_body,
        grid=(N // BMM,),
        in_specs=[_stack_spec(2, BMM)],
        out_specs=pl.BlockSpec((BMM, 64), lambda i: (i, 0)),
        out_shape=jax.ShapeDtypeStruct((N, 64), f32),
    )(parts)
    return out
